# baseline (speedup 1.0000x reference)
import functools

import jax
import jax.numpy as jnp
from jax.experimental import pallas as pl
from jax.experimental.pallas import tpu as pltpu


def _router_body(x_ref, wgt_ref, logits_ref, dw_ref):
    x = x_ref[...]
    wgt = wgt_ref[...]
    logits = jnp.dot(x, wgt, preferred_element_type=jnp.float32)
    logits_ref[...] = logits
    e = logits.shape[1]
    m = jnp.max(logits, axis=1, keepdims=True)
    ex = jnp.exp(logits - m)
    p = ex / jnp.sum(ex, axis=1, keepdims=True)
    idx = jax.lax.broadcasted_iota(jnp.int32, p.shape, 1)
    m1 = jnp.max(p, axis=1, keepdims=True)
    a1 = jnp.min(jnp.where(p == m1, idx, e), axis=1, keepdims=True)
    mask1 = idx == a1
    p2 = jnp.where(mask1, -1.0, p)
    m2 = jnp.max(p2, axis=1, keepdims=True)
    a2 = jnp.min(jnp.where(p2 == m2, idx, e), axis=1, keepdims=True)
    mask2 = idx == a2
    dw_ref[...] = jnp.where(mask1 | mask2, p, 0.0) / (m1 + m2)


def _moe_body(x_ref, w1_ref, w3_ref, w2_ref, dw_ref, out_ref, acc_ref):
    e = pl.program_id(0)
    n_e = pl.num_programs(0)
    fi = pl.program_id(1)
    n_f = pl.num_programs(1)
    step = e * n_f + fi
    x = x_ref[...]
    w1 = w1_ref[0]
    w3 = w3_ref[0]
    w2 = w2_ref[0]
    gate = jnp.dot(x, w1.T, preferred_element_type=jnp.float32)
    up = jnp.dot(x, w3.T, preferred_element_type=jnp.float32)
    act = gate * jax.nn.sigmoid(gate) * up
    y = jnp.dot(act, w2.T, preferred_element_type=jnp.float32)
    dw = dw_ref[...]
    eidx = jax.lax.broadcasted_iota(jnp.int32, dw.shape, 1)
    w_e = jnp.sum(jnp.where(eidx == e, dw, 0.0), axis=1, keepdims=True)
    contrib = y * w_e

    @pl.when(step == 0)
    def _init():
        acc_ref[...] = contrib

    @pl.when(step != 0)
    def _acc():
        acc_ref[...] = acc_ref[...] + contrib

    @pl.when(step == n_e * n_f - 1)
    def _out():
        out_ref[...] = acc_ref[...]


def kernel(hidden_states, Wg, W1, W3, W2):
    b, s, h = hidden_states.shape
    t = b * s
    e, f, _ = W1.shape
    x = hidden_states.reshape(t, h)

    bt_r = 256
    router = pl.pallas_call(
        _router_body,
        grid=(t // bt_r,),
        in_specs=[
            pl.BlockSpec((bt_r, h), lambda i: (i, 0)),
            pl.BlockSpec((h, e), lambda i: (0, 0)),
        ],
        out_specs=[
            pl.BlockSpec((bt_r, e), lambda i: (i, 0)),
            pl.BlockSpec((bt_r, e), lambda i: (i, 0)),
        ],
        out_shape=[
            jax.ShapeDtypeStruct((t, e), jnp.float32),
            jax.ShapeDtypeStruct((t, e), jnp.float32),
        ],
    )
    router_logits, dense_w = router(x, Wg.T)

    nf = 2
    f2 = f // nf
    moe = pl.pallas_call(
        _moe_body,
        grid=(e, nf),
        in_specs=[
            pl.BlockSpec((t, h), lambda ei, fi: (0, 0)),
            pl.BlockSpec((1, f2, h), lambda ei, fi: (ei, fi, 0)),
            pl.BlockSpec((1, f2, h), lambda ei, fi: (ei, fi, 0)),
            pl.BlockSpec((1, h, f2), lambda ei, fi: (ei, 0, fi)),
            pl.BlockSpec((t, e), lambda ei, fi: (0, 0)),
        ],
        out_specs=pl.BlockSpec((t, h), lambda ei, fi: (0, 0)),
        out_shape=jax.ShapeDtypeStruct((t, h), jnp.float32),
        scratch_shapes=[pltpu.VMEM((t, h), jnp.float32)],
        compiler_params=pltpu.CompilerParams(
            dimension_semantics=("arbitrary", "arbitrary"),
        ),
    )
    out = moe(x, W1, W3, W2, dense_w)
    return out.reshape(b, s, h), router_logits


# speedup vs baseline: 1.0656x; 1.0656x over previous
import functools

import jax
import jax.numpy as jnp
from jax.experimental import pallas as pl
from jax.experimental.pallas import tpu as pltpu


def _router_body(x_ref, wgt_ref, logits_ref, dw_ref):
    x = x_ref[...]
    wgt = wgt_ref[...]
    logits = jnp.dot(x, wgt, preferred_element_type=jnp.float32)
    logits_ref[...] = logits
    e = logits.shape[1]
    m = jnp.max(logits, axis=1, keepdims=True)
    ex = jnp.exp(logits - m)
    p = ex / jnp.sum(ex, axis=1, keepdims=True)
    idx = jax.lax.broadcasted_iota(jnp.int32, p.shape, 1)
    m1 = jnp.max(p, axis=1, keepdims=True)
    a1 = jnp.min(jnp.where(p == m1, idx, e), axis=1, keepdims=True)
    mask1 = idx == a1
    p2 = jnp.where(mask1, -1.0, p)
    m2 = jnp.max(p2, axis=1, keepdims=True)
    a2 = jnp.min(jnp.where(p2 == m2, idx, e), axis=1, keepdims=True)
    mask2 = idx == a2
    dw_ref[...] = jnp.where(mask1 | mask2, p, 0.0) / (m1 + m2)


def _moe_body(x_ref, w1_ref, w3_ref, w2_ref, dw_ref, out_ref, acc_ref):
    e = pl.program_id(1)
    n_e = pl.num_programs(1)
    x = x_ref[...]
    w1 = w1_ref[0]
    w3 = w3_ref[0]
    w2 = w2_ref[0]
    gate = jnp.dot(x, w1.T, preferred_element_type=jnp.float32)
    up = jnp.dot(x, w3.T, preferred_element_type=jnp.float32)
    act = gate * jax.nn.sigmoid(gate) * up
    y = jnp.dot(act, w2.T, preferred_element_type=jnp.float32)
    dw = dw_ref[...]
    eidx = jax.lax.broadcasted_iota(jnp.int32, dw.shape, 1)
    w_e = jnp.sum(jnp.where(eidx == e, dw, 0.0), axis=1, keepdims=True)
    contrib = y * w_e

    @pl.when(e == 0)
    def _init():
        acc_ref[...] = contrib

    @pl.when(e != 0)
    def _acc():
        acc_ref[...] = acc_ref[...] + contrib

    @pl.when(e == n_e - 1)
    def _out():
        out_ref[...] = acc_ref[...]


def kernel(hidden_states, Wg, W1, W3, W2):
    b, s, h = hidden_states.shape
    t = b * s
    e, f, _ = W1.shape
    x = hidden_states.reshape(t, h)

    bt_r = 256
    router = pl.pallas_call(
        _router_body,
        grid=(t // bt_r,),
        in_specs=[
            pl.BlockSpec((bt_r, h), lambda i: (i, 0)),
            pl.BlockSpec((h, e), lambda i: (0, 0)),
        ],
        out_specs=[
            pl.BlockSpec((bt_r, e), lambda i: (i, 0)),
            pl.BlockSpec((bt_r, e), lambda i: (i, 0)),
        ],
        out_shape=[
            jax.ShapeDtypeStruct((t, e), jnp.float32),
            jax.ShapeDtypeStruct((t, e), jnp.float32),
        ],
    )
    router_logits, dense_w = router(x, Wg.T)

    bm = 1024
    moe = pl.pallas_call(
        _moe_body,
        grid=(t // bm, e),
        in_specs=[
            pl.BlockSpec((bm, h), lambda ti, ei: (ti, 0)),
            pl.BlockSpec((1, f, h), lambda ti, ei: (ei, 0, 0)),
            pl.BlockSpec((1, f, h), lambda ti, ei: (ei, 0, 0)),
            pl.BlockSpec((1, h, f), lambda ti, ei: (ei, 0, 0)),
            pl.BlockSpec((bm, e), lambda ti, ei: (ti, 0)),
        ],
        out_specs=pl.BlockSpec((bm, h), lambda ti, ei: (ti, 0)),
        out_shape=jax.ShapeDtypeStruct((t, h), jnp.float32),
        scratch_shapes=[pltpu.VMEM((bm, h), jnp.float32)],
        compiler_params=pltpu.CompilerParams(
            dimension_semantics=("arbitrary", "arbitrary"),
        ),
    )
    out = moe(x, W1, W3, W2, dense_w)
    return out.reshape(b, s, h), router_logits
